# baseline (speedup 1.0000x reference)
import jax
import jax.numpy as jnp
from jax import lax
from jax.experimental import pallas as pl
from jax.experimental.pallas import tpu as pltpu
from jax.experimental.pallas import tpu_sc as plsc

_TABLE_ROWS = 1024
_DIM = 768
_BATCH = 16
_SEQ = 1024
_B_PER_W = _BATCH * _SEQ // 32
_CHUNK = 32
_NCHUNK = _B_PER_W // _CHUNK
_NBUF = 4


def _sc_body(table_hbm, idx_hbm, out_hbm, idx_v, *rest):
    bufs = rest[:_NBUF]
    gsems = rest[_NBUF:2 * _NBUF]
    osems = rest[2 * _NBUF:3 * _NBUF]

    wid = lax.axis_index("s") * 2 + lax.axis_index("c")
    row = wid // 2
    col0 = (wid % 2) * _B_PER_W
    pltpu.sync_copy(idx_hbm.at[row, pl.ds(col0, _B_PER_W)], idx_v)

    gathers = [None] * _NCHUNK
    stores = [None] * _NCHUNK

    def start_gather(c):
        b = c % _NBUF
        gathers[c] = pltpu.async_copy(
            table_hbm.at[idx_v.at[pl.ds(c * _CHUNK, _CHUNK)]],
            bufs[b], gsems[b])

    for c in range(min(_NBUF, _NCHUNK)):
        start_gather(c)

    for c in range(_NCHUNK):
        b = c % _NBUF
        gathers[c].wait()
        stores[c] = pltpu.async_copy(
            bufs[b], out_hbm.at[row, pl.ds(col0 + c * _CHUNK, _CHUNK)],
            osems[b])
        nxt = c + _NBUF
        if nxt < _NCHUNK:
            stores[c].wait()
            start_gather(nxt)

    for c in range(max(0, _NCHUNK - _NBUF), _NCHUNK):
        stores[c].wait()


@jax.jit
def _lookup(embeddings, patch_index):
    mesh = plsc.VectorSubcoreMesh(core_axis_name="c", subcore_axis_name="s")
    return pl.kernel(
        _sc_body,
        mesh=mesh,
        out_type=jax.ShapeDtypeStruct((_BATCH, _SEQ, _DIM), jnp.float32),
        scratch_types=(
            [pltpu.VMEM((_B_PER_W,), jnp.int32)]
            + [pltpu.VMEM((_CHUNK, _DIM), jnp.float32)] * _NBUF
            + [pltpu.SemaphoreType.DMA] * (2 * _NBUF)
        ),
    )(embeddings, patch_index)


def kernel(patch_index, embeddings):
    return _lookup(embeddings, patch_index)


# speedup vs baseline: 1.1892x; 1.1892x over previous
import jax
import jax.numpy as jnp
from jax import lax
from jax.experimental import pallas as pl
from jax.experimental.pallas import tpu as pltpu
from jax.experimental.pallas import tpu_sc as plsc

_TABLE_ROWS = 1024
_DIM = 768
_BATCH = 16
_SEQ = 1024
_B_PER_W = _BATCH * _SEQ // 32
_CHUNK = 32
_NCHUNK = _B_PER_W // _CHUNK
_ROWS_PER_TILE = _TABLE_ROWS // 16


def _sc_body(table_hbm, idx_hbm, out_hbm,
             table_sp, idx_v, buf0, buf1,
             rsem0, rsem1, osem0, osem1):
    sid = lax.axis_index("s")
    wid = sid * 2 + lax.axis_index("c")
    row = wid // 2
    col0 = (wid % 2) * _B_PER_W

    pltpu.sync_copy(table_hbm.at[pl.ds(sid * _ROWS_PER_TILE, _ROWS_PER_TILE)],
                    table_sp.at[pl.ds(sid * _ROWS_PER_TILE, _ROWS_PER_TILE)])
    pltpu.sync_copy(idx_hbm.at[row, pl.ds(col0, _B_PER_W)], idx_v)
    plsc.subcore_barrier()

    bufs = (buf0, buf1)
    rsems = (rsem0, rsem1)
    osems = (osem0, osem1)

    def drain_store(b):
        pltpu.make_async_copy(
            out_hbm.at[0, pl.ds(0, _CHUNK)], bufs[b], osems[b]).wait()

    def fill_and_store(t, b):
        c = 2 * t + b
        handles = []
        for g in range(_CHUNK // 16):
            vec = idx_v[pl.ds(c * _CHUNK + g * 16, 16)]
            for j in range(16):
                r = vec[j]
                handles.append(pltpu.async_copy(
                    table_sp.at[pl.ds(r, 1)],
                    bufs[b].at[pl.ds(g * 16 + j, 1)], rsems[b]))
        for h in handles:
            h.wait()
        pltpu.async_copy(
            bufs[b], out_hbm.at[row, pl.ds(col0 + c * _CHUNK, _CHUNK)],
            osems[b])

    def body(t, carry):
        for b in range(2):
            @pl.when(t >= 1)
            def _():
                drain_store(b)
            fill_and_store(t, b)
        return carry

    lax.fori_loop(0, _NCHUNK // 2, body, 0)
    drain_store(0)
    drain_store(1)


@jax.jit
def _lookup(embeddings, patch_index):
    mesh = plsc.VectorSubcoreMesh(core_axis_name="c", subcore_axis_name="s")
    return pl.kernel(
        _sc_body,
        mesh=mesh,
        out_type=jax.ShapeDtypeStruct((_BATCH, _SEQ, _DIM), jnp.float32),
        scratch_types=(
            [pltpu.VMEM_SHARED((_TABLE_ROWS, _DIM), jnp.float32),
             pltpu.VMEM((_B_PER_W,), jnp.int32),
             pltpu.VMEM((_CHUNK, _DIM), jnp.float32),
             pltpu.VMEM((_CHUNK, _DIM), jnp.float32)]
            + [pltpu.SemaphoreType.DMA] * 4
        ),
    )(embeddings, patch_index)


def kernel(patch_index, embeddings):
    return _lookup(embeddings, patch_index)


# speedup vs baseline: 1.1996x; 1.0087x over previous
import jax
import jax.numpy as jnp
from jax import lax
from jax.experimental import pallas as pl
from jax.experimental.pallas import tpu as pltpu
from jax.experimental.pallas import tpu_sc as plsc

_TABLE_ROWS = 1024
_DIM = 768
_BATCH = 16
_SEQ = 1024
_B_PER_W = _BATCH * _SEQ // 32
_CHUNK = 32
_NCHUNK = _B_PER_W // _CHUNK
_ROWS_PER_TILE = _TABLE_ROWS // 16


def _sc_body(table_hbm, idx_hbm, out_hbm,
             table_sp, idx_v, buf0, buf1,
             rsem0, rsem1, osem0, osem1):
    sid = lax.axis_index("s")
    wid = sid * 2 + lax.axis_index("c")
    row = wid // 2
    col0 = (wid % 2) * _B_PER_W

    pltpu.sync_copy(table_hbm.at[pl.ds(sid * _ROWS_PER_TILE, _ROWS_PER_TILE)],
                    table_sp.at[pl.ds(sid * _ROWS_PER_TILE, _ROWS_PER_TILE)])
    pltpu.sync_copy(idx_hbm.at[row, pl.ds(col0, _B_PER_W)], idx_v)
    plsc.subcore_barrier()

    bufs = (buf0, buf1)
    rsems = (rsem0, rsem1)
    osems = (osem0, osem1)

    def drain_store(b):
        pltpu.make_async_copy(
            out_hbm.at[0, pl.ds(0, _CHUNK)], bufs[b], osems[b]).wait()

    def fill_and_store(t, b):
        c = 2 * t + b
        for g in range(_CHUNK // 16):
            vec = idx_v[pl.ds(c * _CHUNK + g * 16, 16)]
            for j in range(16):
                r = vec[j]
                pltpu.async_copy(
                    table_sp.at[pl.ds(r, 1)],
                    bufs[b].at[pl.ds(g * 16 + j, 1)], rsems[b])
        pltpu.make_async_copy(
            out_hbm.at[0, pl.ds(0, _CHUNK)], bufs[b], rsems[b]).wait()
        pltpu.async_copy(
            bufs[b], out_hbm.at[row, pl.ds(col0 + c * _CHUNK, _CHUNK)],
            osems[b])

    def body(t, carry):
        for b in range(2):
            @pl.when(t >= 1)
            def _():
                drain_store(b)
            fill_and_store(t, b)
        return carry

    lax.fori_loop(0, _NCHUNK // 2, body, 0)
    drain_store(0)
    drain_store(1)


@jax.jit
def _lookup(embeddings, patch_index):
    mesh = plsc.VectorSubcoreMesh(core_axis_name="c", subcore_axis_name="s")
    return pl.kernel(
        _sc_body,
        mesh=mesh,
        out_type=jax.ShapeDtypeStruct((_BATCH, _SEQ, _DIM), jnp.float32),
        scratch_types=(
            [pltpu.VMEM_SHARED((_TABLE_ROWS, _DIM), jnp.float32),
             pltpu.VMEM((_B_PER_W,), jnp.int32),
             pltpu.VMEM((_CHUNK, _DIM), jnp.float32),
             pltpu.VMEM((_CHUNK, _DIM), jnp.float32)]
            + [pltpu.SemaphoreType.DMA] * 4
        ),
    )(embeddings, patch_index)


def kernel(patch_index, embeddings):
    return _lookup(embeddings, patch_index)
